# baseline (speedup 1.0000x reference)
import jax
import jax.numpy as jnp
from jax.experimental import pallas as pl

_BLK = 512


def _add_pe_kernel(x_ref, pe_ref, o_ref):
    o_ref[...] = x_ref[...] + pe_ref[...][None, :, :]


def kernel(x, pe):
    B, S, E = x.shape
    blk = min(_BLK, S)
    ec = E // 2
    grid = (S // blk, 2)
    return pl.pallas_call(
        _add_pe_kernel,
        grid=grid,
        in_specs=[
            pl.BlockSpec((B, blk, ec), lambda i, j: (0, i, j)),
            pl.BlockSpec((blk, ec), lambda i, j: (i, j)),
        ],
        out_specs=pl.BlockSpec((B, blk, ec), lambda i, j: (0, i, j)),
        out_shape=jax.ShapeDtypeStruct((B, S, E), x.dtype),
    )(x, pe)


# speedup vs baseline: 1.0279x; 1.0279x over previous
import jax
import jax.numpy as jnp
from jax.experimental import pallas as pl

_BLK = 512


def _add_pe_kernel(x_ref, pe_ref, o_ref):
    o_ref[...] = x_ref[...] + pe_ref[...][None, :, :]


def kernel(x, pe):
    B, S, E = x.shape
    blk = min(_BLK, S)
    grid = (S // blk,)
    return pl.pallas_call(
        _add_pe_kernel,
        grid=grid,
        in_specs=[
            pl.BlockSpec((B, blk, E), lambda i: (0, i, 0)),
            pl.BlockSpec((blk, E), lambda i: (i, 0)),
        ],
        out_specs=pl.BlockSpec((B, blk, E), lambda i: (0, i, 0)),
        out_shape=jax.ShapeDtypeStruct((B, S, E), x.dtype),
    )(x, pe)
